# Initial kernel scaffold; baseline (speedup 1.0000x reference)
import jax
import jax.numpy as jnp
from jax.experimental import pallas as pl


def kernel(x, edge_index, W1, b1, W2, b2, W3, b3):
    raise NotImplementedError("write your pallas kernel here")


# speedup vs baseline: 5.0956x; 5.0956x over previous
import functools

import jax
import jax.numpy as jnp
from jax import lax
from jax.experimental import pallas as pl
from jax.experimental.pallas import tpu as pltpu
from jax.experimental.pallas import tpu_sc as plsc

N_NODES = 10000
N_EDGES = 160000
D = 256
HALF = 128

NUM_CORES = 2
NUM_TILES = 16

CHUNK = 125
N_CHUNKS = N_EDGES // CHUNK
CHUNKS_PER_TILE = N_CHUNKS // NUM_TILES
ROWS_PER_TILE = 632
N_PAD = ROWS_PER_TILE * NUM_TILES

R_BLK = 2000


def _elu(v):
    return jnp.where(v > 0, v, jnp.exp(v) - 1.0)


def _mm_first_body(x_ref, w_ref, o_ref):
    xb = x_ref[...]
    o_ref[0] = jnp.dot(xb, w_ref[:, :HALF], preferred_element_type=jnp.float32)
    o_ref[1] = jnp.dot(xb, w_ref[:, HALF:], preferred_element_type=jnp.float32)


def _mm_first(x, w):
    grid = (N_NODES // R_BLK,)
    return pl.pallas_call(
        _mm_first_body,
        grid=grid,
        in_specs=[
            pl.BlockSpec((R_BLK, D), lambda i: (i, 0)),
            pl.BlockSpec((D, D), lambda i: (0, 0)),
        ],
        out_specs=pl.BlockSpec((NUM_CORES, R_BLK, HALF), lambda i: (0, i, 0)),
        out_shape=jax.ShapeDtypeStruct((NUM_CORES, N_NODES, HALF), jnp.float32),
    )(x, w)


def _mm_mid_body(y_ref, a_ref, b_ref, w_ref, o_ref):
    h0 = _elu(y_ref[0] + a_ref[0] + b_ref[0])
    h1 = _elu(y_ref[1] + a_ref[1] + b_ref[1])
    r = jnp.dot(h0, w_ref[:HALF, :], preferred_element_type=jnp.float32)
    r = r + jnp.dot(h1, w_ref[HALF:, :], preferred_element_type=jnp.float32)
    o_ref[0] = r[:, :HALF]
    o_ref[1] = r[:, HALF:]


def _mm_mid(y, agg, b2d, w):
    grid = (N_NODES // R_BLK,)
    return pl.pallas_call(
        _mm_mid_body,
        grid=grid,
        in_specs=[
            pl.BlockSpec((NUM_CORES, R_BLK, HALF), lambda i: (0, i, 0)),
            pl.BlockSpec((NUM_CORES, R_BLK, HALF), lambda i: (0, i, 0)),
            pl.BlockSpec((NUM_CORES, 1, HALF), lambda i: (0, 0, 0)),
            pl.BlockSpec((D, D), lambda i: (0, 0)),
        ],
        out_specs=pl.BlockSpec((NUM_CORES, R_BLK, HALF), lambda i: (0, i, 0)),
        out_shape=jax.ShapeDtypeStruct((NUM_CORES, N_NODES, HALF), jnp.float32),
    )(y, agg, b2d, w)


def _final_body(y_ref, a_ref, b_ref, o_ref):
    o_ref[:, :HALF] = y_ref[0] + a_ref[0] + b_ref[0]
    o_ref[:, HALF:] = y_ref[1] + a_ref[1] + b_ref[1]


def _final(y, agg, b2d):
    grid = (N_NODES // R_BLK,)
    return pl.pallas_call(
        _final_body,
        grid=grid,
        in_specs=[
            pl.BlockSpec((NUM_CORES, R_BLK, HALF), lambda i: (0, i, 0)),
            pl.BlockSpec((NUM_CORES, R_BLK, HALF), lambda i: (0, i, 0)),
            pl.BlockSpec((NUM_CORES, 1, HALF), lambda i: (0, 0, 0)),
        ],
        out_specs=pl.BlockSpec((R_BLK, D), lambda i: (i, 0)),
        out_shape=jax.ShapeDtypeStruct((N_NODES, D), jnp.float32),
    )(y, agg, b2d)


def _sc_agg_body(y_hbm, gsrc_hbm, dst_hbm, zeros_hbm, out_hbm,
                 src_v, dst_v, rows_v, acc_sh, sem):
    c = lax.axis_index("c")
    s = lax.axis_index("s")

    pltpu.sync_copy(zeros_hbm, acc_sh.at[pl.ds(s * ROWS_PER_TILE, ROWS_PER_TILE)])

    base = s * CHUNKS_PER_TILE
    pltpu.sync_copy(gsrc_hbm.at[c].at[pl.ds(base, CHUNKS_PER_TILE)], src_v)
    pltpu.sync_copy(dst_hbm.at[pl.ds(base, CHUNKS_PER_TILE)], dst_v)

    plsc.subcore_barrier()

    def body(j, carry):
        pltpu.async_copy(y_hbm.at[src_v.at[j]], rows_v, sem).wait()
        pltpu.sync_copy(rows_v, acc_sh.at[dst_v.at[j]], add=True)
        return carry

    lax.fori_loop(0, CHUNKS_PER_TILE, body, 0)

    plsc.subcore_barrier()

    r0 = s * ROWS_PER_TILE
    pltpu.sync_copy(
        acc_sh.at[pl.ds(r0, ROWS_PER_TILE)],
        out_hbm.at[c].at[pl.ds(r0, ROWS_PER_TILE)],
    )


@functools.lru_cache(maxsize=1)
def _make_sc_agg_kernel():
    return pl.kernel(
        _sc_agg_body,
        out_type=jax.ShapeDtypeStruct((NUM_CORES, N_PAD, HALF), jnp.float32),
        mesh=plsc.VectorSubcoreMesh(
            core_axis_name="c", subcore_axis_name="s",
            num_cores=NUM_CORES, num_subcores=NUM_TILES,
        ),
        scratch_types=[
            pltpu.VMEM((CHUNKS_PER_TILE, CHUNK), jnp.int32),
            pltpu.VMEM((CHUNKS_PER_TILE, CHUNK), jnp.int32),
            pltpu.VMEM((CHUNK, HALF), jnp.float32),
            pltpu.VMEM_SHARED((N_PAD, HALF), jnp.float32),
            pltpu.SemaphoreType.DMA,
        ],
    )


def _sc_agg(y, gsrc, dstc, zeros):
    yf = y.reshape(NUM_CORES * N_NODES, HALF)
    return _make_sc_agg_kernel()(yf, gsrc, dstc, zeros)[:, :N_NODES, :]


def kernel(x, edge_index, W1, b1, W2, b2, W3, b3):
    ei = edge_index.astype(jnp.int32)
    src = ei[0]
    dst = ei[1]

    gsrc = jnp.stack([src, src + N_NODES]).reshape(NUM_CORES, N_CHUNKS, CHUNK)
    dstc = dst.reshape(N_CHUNKS, CHUNK)
    zeros = jnp.zeros((ROWS_PER_TILE, HALF), jnp.float32)

    b1h = b1.reshape(NUM_CORES, 1, HALF)
    b2h = b2.reshape(NUM_CORES, 1, HALF)
    b3h = b3.reshape(NUM_CORES, 1, HALF)

    y1 = _mm_first(x, W1)
    a1 = _sc_agg(y1, gsrc, dstc, zeros)
    y2 = _mm_mid(y1, a1, b1h, W2)
    a2 = _sc_agg(y2, gsrc, dstc, zeros)
    y3 = _mm_mid(y2, a2, b2h, W3)
    a3 = _sc_agg(y3, gsrc, dstc, zeros)
    return _final(y3, a3, b3h)
